# baseline (speedup 1.0000x reference)
import functools

import jax
import jax.numpy as jnp
from jax import lax
from jax.experimental import pallas as pl
from jax.experimental.pallas import tpu as pltpu
from jax.experimental.pallas import tpu_sc as plsc

N = 10000
E = 320000
D = 128
D_OUT = 64
BN_EPS = 1e-5

NC = 2
NS = 16
NW = NC * NS
C = 128
NBUF = 2
QC = 16
SEG = 5
K = QC * SEG
E_PAD = NW * K * C
AG_ROWS = 10112
ZROWS = 632


def _sc_aggregate(x, src_kc, dst_kc, zeros_blk):
    mesh = plsc.VectorSubcoreMesh(core_axis_name="c", subcore_axis_name="s")

    @functools.partial(
        pl.kernel,
        out_type=jax.ShapeDtypeStruct((NC, AG_ROWS, D), jnp.float32),
        mesh=mesh,
        scratch_types=[
            pltpu.VMEM((QC, C), jnp.int32),
            pltpu.VMEM((QC, C), jnp.int32),
            [pltpu.VMEM((C, D), jnp.float32) for _ in range(NBUF)],
            pltpu.VMEM_SHARED((AG_ROWS, D), jnp.float32),
            [pltpu.SemaphoreType.DMA for _ in range(NBUF)],
            [pltpu.SemaphoreType.DMA for _ in range(NBUF)],
        ],
    )
    def body(x_hbm, src_hbm, dst_hbm, z_hbm, out_hbm,
             src_v, dst_v, rows_bufs, aggr_sh, gsems, ssems):
        c = lax.axis_index("c")
        s = lax.axis_index("s")
        wid = s * NC + c

        def gather(m, t):
            pltpu.async_copy(x_hbm.at[src_v.at[m]], rows_bufs[t], gsems[t])

        def gather_wait(m, t):
            pltpu.make_async_copy(x_hbm.at[src_v.at[m]], rows_bufs[t],
                                  gsems[t]).wait()

        def scatter(m, t):
            pltpu.async_copy(rows_bufs[t], aggr_sh.at[dst_v.at[m]], ssems[t],
                             add=True)

        def scatter_wait(m, t):
            pltpu.make_async_copy(rows_bufs[t], aggr_sh.at[dst_v.at[m]],
                                  ssems[t]).wait()

        pltpu.sync_copy(z_hbm, aggr_sh.at[pl.ds(s * ZROWS, ZROWS)])
        plsc.subcore_barrier()

        for q in range(SEG):
            pltpu.sync_copy(src_hbm.at[wid].at[pl.ds(q * QC, QC)], src_v)
            pltpu.sync_copy(dst_hbm.at[wid].at[pl.ds(q * QC, QC)], dst_v)
            gather(0, 0)
            gather(1, 1)

            gather_wait(0, 0)
            scatter(0, 0)
            gather_wait(1, 1)
            scatter(1, 1)
            scatter_wait(0, 0)
            gather(2, 0)

            def group(g, carry):
                m = 2 * g
                gather_wait(m, 0)
                scatter(m, 0)
                scatter_wait(m - 1, 1)
                gather(m + 1, 1)
                gather_wait(m + 1, 1)
                scatter(m + 1, 1)
                scatter_wait(m, 0)
                gather(m + 2, 0)
                return carry

            lax.fori_loop(1, QC // 2 - 1, group, 0)

            gather_wait(QC - 2, 0)
            scatter(QC - 2, 0)
            scatter_wait(QC - 3, 1)
            gather(QC - 1, 1)
            gather_wait(QC - 1, 1)
            scatter(QC - 1, 1)
            scatter_wait(QC - 2, 0)
            scatter_wait(QC - 1, 1)
        plsc.subcore_barrier()

        pltpu.sync_copy(aggr_sh.at[pl.ds(s * ZROWS, ZROWS)],
                        out_hbm.at[c].at[pl.ds(s * ZROWS, ZROWS)])

    return body(x, src_kc, dst_kc, zeros_blk)


def _dense_bn_layer(x, aggr, W1, b1, W2, b2, gamma, beta):
    def body(x_ref, a_ref, w1_ref, b1_ref, w2_ref, b2_ref, g_ref, be_ref,
             o_ref):
        h = x_ref[...] + a_ref[0, :N] + a_ref[1, :N]
        y = jnp.dot(h, w1_ref[...], preferred_element_type=jnp.float32)
        y = jnp.maximum(y + b1_ref[...], 0.0)
        y = jnp.dot(y, w2_ref[...], preferred_element_type=jnp.float32)
        y = y + b2_ref[...]
        mean = jnp.mean(y, axis=0, keepdims=True)
        var = jnp.mean(jnp.square(y - mean), axis=0, keepdims=True)
        yn = (y - mean) * lax.rsqrt(var + BN_EPS) * g_ref[...] + be_ref[...]
        o_ref[...] = jnp.maximum(yn, 0.0) + x_ref[...]

    return pl.pallas_call(
        body,
        out_shape=jax.ShapeDtypeStruct((N, D), jnp.float32),
    )(x, aggr, W1, b1.reshape(1, D), W2, b2.reshape(1, D),
      gamma.reshape(1, D), beta.reshape(1, D))


def _dense_final_layer(x, aggr, W1, b1, W2, b2):
    def body(x_ref, a_ref, w1_ref, b1_ref, w2_ref, b2_ref, o_ref):
        h = x_ref[...] + a_ref[0, :N] + a_ref[1, :N]
        y = jnp.dot(h, w1_ref[...], preferred_element_type=jnp.float32)
        y = jnp.maximum(y + b1_ref[...], 0.0)
        y = jnp.dot(y, w2_ref[...], preferred_element_type=jnp.float32)
        y = y + b2_ref[...]
        m = jnp.max(y, axis=1, keepdims=True)
        shifted = y - m
        lse = jnp.log(jnp.sum(jnp.exp(shifted), axis=1, keepdims=True))
        o_ref[...] = shifted - lse

    return pl.pallas_call(
        body,
        out_shape=jax.ShapeDtypeStruct((N, D_OUT), jnp.float32),
    )(x, aggr, W1, b1.reshape(1, D), W2, b2.reshape(1, D_OUT))


def kernel(x, edge_index,
           W1_0, b1_0, W2_0, b2_0,
           W1_1, b1_1, W2_1, b2_1,
           W1_2, b1_2, W2_2, b2_2,
           gamma0, beta0, gamma1, beta1):
    pad = E_PAD - E
    ar = jnp.arange(pad, dtype=jnp.int32)
    src = jnp.concatenate([edge_index[0], ar % N])
    dst = jnp.concatenate([edge_index[1], N + ar % (AG_ROWS - N)])
    src_kc = src.reshape(NW, K, C)
    dst_kc = dst.reshape(NW, K, C)
    zeros_blk = jnp.zeros((ZROWS, D), jnp.float32)

    a0 = _sc_aggregate(x, src_kc, dst_kc, zeros_blk)
    x1 = _dense_bn_layer(x, a0, W1_0, b1_0, W2_0, b2_0, gamma0, beta0)
    a1 = _sc_aggregate(x1, src_kc, dst_kc, zeros_blk)
    x2 = _dense_bn_layer(x1, a1, W1_1, b1_1, W2_1, b2_1, gamma1, beta1)
    a2 = _sc_aggregate(x2, src_kc, dst_kc, zeros_blk)
    return _dense_final_layer(x2, a2, W1_2, b1_2, W2_2, b2_2)


# speedup vs baseline: 1.2257x; 1.2257x over previous
import functools

import jax
import jax.numpy as jnp
from jax import lax
from jax.experimental import pallas as pl
from jax.experimental.pallas import tpu as pltpu
from jax.experimental.pallas import tpu_sc as plsc

N = 10000
E = 320000
D = 128
D_OUT = 64
BN_EPS = 1e-5

NC = 2
NS = 16
NW = NC * NS
C = 128
NBUF = 2
QC = 40
SEG = 2
K = QC * SEG
E_PAD = NW * K * C
AG_ROWS = 10112
ZROWS = 632


def _sc_aggregate(x, src_kc, dst_kc, zeros_blk):
    mesh = plsc.VectorSubcoreMesh(core_axis_name="c", subcore_axis_name="s")

    @functools.partial(
        pl.kernel,
        out_type=jax.ShapeDtypeStruct((NC, AG_ROWS, D), jnp.float32),
        mesh=mesh,
        scratch_types=[
            pltpu.VMEM((QC, C), jnp.int32),
            pltpu.VMEM((QC, C), jnp.int32),
            [pltpu.VMEM((C, D), jnp.float32) for _ in range(NBUF)],
            pltpu.VMEM_SHARED((AG_ROWS, D), jnp.float32),
            [pltpu.SemaphoreType.DMA for _ in range(NBUF)],
        ],
    )
    def body(x_hbm, src_hbm, dst_hbm, z_hbm, out_hbm,
             src_v, dst_v, rows_bufs, aggr_sh, gsems):
        c = lax.axis_index("c")
        s = lax.axis_index("s")
        wid = s * NC + c

        def gather(m, t):
            pltpu.async_copy(x_hbm.at[src_v.at[m]], rows_bufs[t], gsems[t])

        def gather_wait(m, t):
            pltpu.make_async_copy(x_hbm.at[src_v.at[m]], rows_bufs[t],
                                  gsems[t]).wait()

        pltpu.sync_copy(z_hbm, aggr_sh.at[pl.ds(s * ZROWS, ZROWS)])
        plsc.subcore_barrier()

        for q in range(SEG):
            pltpu.sync_copy(src_hbm.at[wid].at[pl.ds(q * QC, QC)], src_v)
            pltpu.sync_copy(dst_hbm.at[wid].at[pl.ds(q * QC, QC)], dst_v)
            for t in range(NBUF):
                gather(t, t)

            def group(base, carry):
                for t in range(NBUF):
                    m = base + t
                    gather_wait(m, t)
                    pltpu.sync_copy(rows_bufs[t], aggr_sh.at[dst_v.at[m]],
                                    add=True)

                    @pl.when(m + NBUF < QC)
                    def _():
                        gather(m + NBUF, t)
                return carry

            lax.fori_loop(0, QC // NBUF,
                          lambda i, c_: group(NBUF * i, c_), 0)
        plsc.subcore_barrier()

        pltpu.sync_copy(aggr_sh.at[pl.ds(s * ZROWS, ZROWS)],
                        out_hbm.at[c].at[pl.ds(s * ZROWS, ZROWS)])

    return body(x, src_kc, dst_kc, zeros_blk)


def _dense_bn_layer(x, aggr, W1, b1, W2, b2, gamma, beta):
    def body(x_ref, a_ref, w1_ref, b1_ref, w2_ref, b2_ref, g_ref, be_ref,
             o_ref):
        h = x_ref[...] + a_ref[0, :N] + a_ref[1, :N]
        y = jnp.dot(h, w1_ref[...], preferred_element_type=jnp.float32)
        y = jnp.maximum(y + b1_ref[...], 0.0)
        y = jnp.dot(y, w2_ref[...], preferred_element_type=jnp.float32)
        y = y + b2_ref[...]
        mean = jnp.mean(y, axis=0, keepdims=True)
        var = jnp.mean(jnp.square(y - mean), axis=0, keepdims=True)
        yn = (y - mean) * lax.rsqrt(var + BN_EPS) * g_ref[...] + be_ref[...]
        o_ref[...] = jnp.maximum(yn, 0.0) + x_ref[...]

    return pl.pallas_call(
        body,
        out_shape=jax.ShapeDtypeStruct((N, D), jnp.float32),
    )(x, aggr, W1, b1.reshape(1, D), W2, b2.reshape(1, D),
      gamma.reshape(1, D), beta.reshape(1, D))


def _dense_final_layer(x, aggr, W1, b1, W2, b2):
    def body(x_ref, a_ref, w1_ref, b1_ref, w2_ref, b2_ref, o_ref):
        h = x_ref[...] + a_ref[0, :N] + a_ref[1, :N]
        y = jnp.dot(h, w1_ref[...], preferred_element_type=jnp.float32)
        y = jnp.maximum(y + b1_ref[...], 0.0)
        y = jnp.dot(y, w2_ref[...], preferred_element_type=jnp.float32)
        y = y + b2_ref[...]
        m = jnp.max(y, axis=1, keepdims=True)
        shifted = y - m
        lse = jnp.log(jnp.sum(jnp.exp(shifted), axis=1, keepdims=True))
        o_ref[...] = shifted - lse

    return pl.pallas_call(
        body,
        out_shape=jax.ShapeDtypeStruct((N, D_OUT), jnp.float32),
    )(x, aggr, W1, b1.reshape(1, D), W2, b2.reshape(1, D_OUT))


def kernel(x, edge_index,
           W1_0, b1_0, W2_0, b2_0,
           W1_1, b1_1, W2_1, b2_1,
           W1_2, b1_2, W2_2, b2_2,
           gamma0, beta0, gamma1, beta1):
    pad = E_PAD - E
    ar = jnp.arange(pad, dtype=jnp.int32)
    src = jnp.concatenate([edge_index[0], ar % N])
    dst = jnp.concatenate([edge_index[1], N + ar % (AG_ROWS - N)])
    src_kc = src.reshape(NW, K, C)
    dst_kc = dst.reshape(NW, K, C)
    zeros_blk = jnp.zeros((ZROWS, D), jnp.float32)

    a0 = _sc_aggregate(x, src_kc, dst_kc, zeros_blk)
    x1 = _dense_bn_layer(x, a0, W1_0, b1_0, W2_0, b2_0, gamma0, beta0)
    a1 = _sc_aggregate(x1, src_kc, dst_kc, zeros_blk)
    x2 = _dense_bn_layer(x1, a1, W1_1, b1_1, W2_1, b2_1, gamma1, beta1)
    a2 = _sc_aggregate(x2, src_kc, dst_kc, zeros_blk)
    return _dense_final_layer(x2, a2, W1_2, b1_2, W2_2, b2_2)


# speedup vs baseline: 1.2756x; 1.0408x over previous
import functools

import jax
import jax.numpy as jnp
from jax import lax
from jax.experimental import pallas as pl
from jax.experimental.pallas import tpu as pltpu
from jax.experimental.pallas import tpu_sc as plsc

N = 10000
E = 320000
D = 128
D_OUT = 64
BN_EPS = 1e-5

NC = 2
NS = 16
NW = NC * NS
C = 64
NBUF = 4
QC = 40
SEG = 4
K = QC * SEG
E_PAD = NW * K * C
AG_ROWS = 10112
ZROWS = 632


def _sc_aggregate(x, src_kc, dst_kc, zeros_blk):
    mesh = plsc.VectorSubcoreMesh(core_axis_name="c", subcore_axis_name="s")

    @functools.partial(
        pl.kernel,
        out_type=jax.ShapeDtypeStruct((NC, AG_ROWS, D), jnp.float32),
        mesh=mesh,
        scratch_types=[
            pltpu.VMEM((QC, C), jnp.int32),
            pltpu.VMEM((QC, C), jnp.int32),
            [pltpu.VMEM((C, D), jnp.float32) for _ in range(NBUF)],
            pltpu.VMEM_SHARED((AG_ROWS, D), jnp.float32),
            [pltpu.SemaphoreType.DMA for _ in range(NBUF)],
        ],
    )
    def body(x_hbm, src_hbm, dst_hbm, z_hbm, out_hbm,
             src_v, dst_v, rows_bufs, aggr_sh, gsems):
        c = lax.axis_index("c")
        s = lax.axis_index("s")
        wid = s * NC + c

        def gather(m, t):
            pltpu.async_copy(x_hbm.at[src_v.at[m]], rows_bufs[t], gsems[t])

        def gather_wait(m, t):
            pltpu.make_async_copy(x_hbm.at[src_v.at[m]], rows_bufs[t],
                                  gsems[t]).wait()

        pltpu.sync_copy(z_hbm, aggr_sh.at[pl.ds(s * ZROWS, ZROWS)])
        plsc.subcore_barrier()

        for q in range(SEG):
            pltpu.sync_copy(src_hbm.at[wid].at[pl.ds(q * QC, QC)], src_v)
            pltpu.sync_copy(dst_hbm.at[wid].at[pl.ds(q * QC, QC)], dst_v)
            for t in range(NBUF):
                gather(t, t)

            def group(base, carry):
                for t in range(NBUF):
                    m = base + t
                    gather_wait(m, t)
                    pltpu.sync_copy(rows_bufs[t], aggr_sh.at[dst_v.at[m]],
                                    add=True)

                    @pl.when(m + NBUF < QC)
                    def _():
                        gather(m + NBUF, t)
                return carry

            lax.fori_loop(0, QC // NBUF,
                          lambda i, c_: group(NBUF * i, c_), 0)
        plsc.subcore_barrier()

        pltpu.sync_copy(aggr_sh.at[pl.ds(s * ZROWS, ZROWS)],
                        out_hbm.at[c].at[pl.ds(s * ZROWS, ZROWS)])

    return body(x, src_kc, dst_kc, zeros_blk)


def _dense_bn_layer(x, aggr, W1, b1, W2, b2, gamma, beta):
    def body(x_ref, a_ref, w1_ref, b1_ref, w2_ref, b2_ref, g_ref, be_ref,
             o_ref):
        h = x_ref[...] + a_ref[0, :N] + a_ref[1, :N]
        y = jnp.dot(h, w1_ref[...], preferred_element_type=jnp.float32)
        y = jnp.maximum(y + b1_ref[...], 0.0)
        y = jnp.dot(y, w2_ref[...], preferred_element_type=jnp.float32)
        y = y + b2_ref[...]
        mean = jnp.mean(y, axis=0, keepdims=True)
        var = jnp.mean(jnp.square(y - mean), axis=0, keepdims=True)
        yn = (y - mean) * lax.rsqrt(var + BN_EPS) * g_ref[...] + be_ref[...]
        o_ref[...] = jnp.maximum(yn, 0.0) + x_ref[...]

    return pl.pallas_call(
        body,
        out_shape=jax.ShapeDtypeStruct((N, D), jnp.float32),
    )(x, aggr, W1, b1.reshape(1, D), W2, b2.reshape(1, D),
      gamma.reshape(1, D), beta.reshape(1, D))


def _dense_final_layer(x, aggr, W1, b1, W2, b2):
    def body(x_ref, a_ref, w1_ref, b1_ref, w2_ref, b2_ref, o_ref):
        h = x_ref[...] + a_ref[0, :N] + a_ref[1, :N]
        y = jnp.dot(h, w1_ref[...], preferred_element_type=jnp.float32)
        y = jnp.maximum(y + b1_ref[...], 0.0)
        y = jnp.dot(y, w2_ref[...], preferred_element_type=jnp.float32)
        y = y + b2_ref[...]
        m = jnp.max(y, axis=1, keepdims=True)
        shifted = y - m
        lse = jnp.log(jnp.sum(jnp.exp(shifted), axis=1, keepdims=True))
        o_ref[...] = shifted - lse

    return pl.pallas_call(
        body,
        out_shape=jax.ShapeDtypeStruct((N, D_OUT), jnp.float32),
    )(x, aggr, W1, b1.reshape(1, D), W2, b2.reshape(1, D_OUT))


def kernel(x, edge_index,
           W1_0, b1_0, W2_0, b2_0,
           W1_1, b1_1, W2_1, b2_1,
           W1_2, b1_2, W2_2, b2_2,
           gamma0, beta0, gamma1, beta1):
    pad = E_PAD - E
    ar = jnp.arange(pad, dtype=jnp.int32)
    src = jnp.concatenate([edge_index[0], ar % N])
    dst = jnp.concatenate([edge_index[1], N + ar % (AG_ROWS - N)])
    src_kc = src.reshape(NW, K, C)
    dst_kc = dst.reshape(NW, K, C)
    zeros_blk = jnp.zeros((ZROWS, D), jnp.float32)

    a0 = _sc_aggregate(x, src_kc, dst_kc, zeros_blk)
    x1 = _dense_bn_layer(x, a0, W1_0, b1_0, W2_0, b2_0, gamma0, beta0)
    a1 = _sc_aggregate(x1, src_kc, dst_kc, zeros_blk)
    x2 = _dense_bn_layer(x1, a1, W1_1, b1_1, W2_1, b2_1, gamma1, beta1)
    a2 = _sc_aggregate(x2, src_kc, dst_kc, zeros_blk)
    return _dense_final_layer(x2, a2, W1_2, b1_2, W2_2, b2_2)
